# baseline (speedup 1.0000x reference)
import functools

import numpy as np
import jax
import jax.numpy as jnp
from jax import lax
from jax.experimental import pallas as pl
from jax.experimental.pallas import tpu as pltpu
from jax.experimental.pallas import tpu_sc as plsc

_VOCAB = 100000
_D = 1024
_HD = 64
_THETA = 10000.0
_B = 4
_S = 2048
_MIN = float(np.finfo(np.float32).min)

_NC = 2
_NS = 16
_NW = _NC * _NS
_TOK = _B * _S
_RPW = _TOK // _NW
_CH = 32
_NCH = _RPW // _CH


@functools.cache
def _make_sc_gather():
    mesh = plsc.VectorSubcoreMesh(core_axis_name="c", subcore_axis_name="s")

    @functools.partial(
        pl.kernel,
        mesh=mesh,
        out_type=jax.ShapeDtypeStruct((_TOK, _D), jnp.float32),
        scratch_types=[
            pltpu.VMEM((_NCH, _CH), jnp.int32),
            pltpu.VMEM((_CH, _D), jnp.float32),
            pltpu.VMEM((_CH, _D), jnp.float32),
            pltpu.SemaphoreType.DMA,
            pltpu.SemaphoreType.DMA,
            pltpu.SemaphoreType.DMA,
            pltpu.SemaphoreType.DMA,
        ],
    )
    def _sc_gather(table, ids, out, idx_v, buf_a, buf_b, gsem_a, gsem_b, wsem_a, wsem_b):
        wid = lax.axis_index("s") * _NC + lax.axis_index("c")
        base = pl.multiple_of(wid * _RPW, _RPW)
        pltpu.sync_copy(ids.at[wid], idx_v)

        def gather(c, buf, gsem):
            pltpu.async_copy(table.at[idx_v.at[c]], buf, gsem)

        def write(c, buf, wsem):
            pltpu.async_copy(buf, out.at[pl.ds(base + c * _CH, _CH)], wsem)

        def wait_gather(buf, gsem):
            pltpu.make_async_copy(table.at[idx_v.at[0]], buf, gsem).wait()

        def wait_write(buf, wsem):
            pltpu.make_async_copy(buf, out.at[pl.ds(base, _CH)], wsem).wait()

        gather(0, buf_a, gsem_a)
        gather(1, buf_b, gsem_b)
        npair = _NCH // 2

        def body(t, carry):
            c = pl.multiple_of(t * 2, 2)
            wait_gather(buf_a, gsem_a)
            write(c, buf_a, wsem_a)
            wait_gather(buf_b, gsem_b)
            write(c + 1, buf_b, wsem_b)

            @pl.when(t + 1 < npair)
            def _():
                wait_write(buf_a, wsem_a)
                gather(c + 2, buf_a, gsem_a)
                wait_write(buf_b, wsem_b)
                gather(c + 3, buf_b, gsem_b)

            return carry

        lax.fori_loop(0, npair, body, 0)
        wait_write(buf_a, wsem_a)
        wait_write(buf_b, wsem_b)

    return _sc_gather


_RBLK = 512
_NRB = _S // _RBLK

_inv_half = 1.0 / (_THETA ** (np.arange(0, _HD, 2, dtype=np.float32) / np.float32(_HD)))
_INV2 = np.concatenate([_inv_half, _inv_half]).reshape(_HD, 1).astype(np.float32)


def _mask_rope_body(am_ref, pos_ref, inv_ref, mask_hbm, cos_ref, sin_ref, bufs, sems):
    r = pl.program_id(0)
    row = lax.broadcasted_iota(jnp.int32, (_RBLK, _S), 0)
    col = lax.broadcasted_iota(jnp.int32, (_RBLK, _S), 1)
    causal = jnp.where(col - row > r * _RBLK, _MIN, 0.0)
    s = r % 2
    for b in range(_B):
        i = s * _B + b

        @pl.when(r >= 2)
        def _wait(i=i, b=b):
            pltpu.make_async_copy(
                bufs.at[i], mask_hbm.at[b, 0, pl.ds(0, _RBLK)], sems.at[i]
            ).wait()

        pb = jnp.where(am_ref[b, :][None, :] == 0.0, _MIN, 0.0)
        bufs[i] = jnp.minimum(causal, pb)
        pltpu.async_copy(
            bufs.at[i], mask_hbm.at[b, 0, pl.ds(r * _RBLK, _RBLK)], sems.at[i]
        )

    @pl.when(r == _NRB - 1)
    def _drain():
        for i in range(2 * _B):
            pltpu.make_async_copy(
                bufs.at[i], mask_hbm.at[0, 0, pl.ds(0, _RBLK)], sems.at[i]
            ).wait()

    @pl.when(r == 0)
    def _():
        pos_f = pos_ref[...].astype(jnp.float32)
        emb_t = inv_ref[...] * pos_f
        cos_ref[0] = jnp.cos(emb_t)
        sin_ref[0] = jnp.sin(emb_t)


def _mask_rope(attention_mask, position_ids, inv2):
    return pl.pallas_call(
        _mask_rope_body,
        grid=(_NRB,),
        in_specs=[
            pl.BlockSpec((_B, _S), lambda r: (0, 0)),
            pl.BlockSpec((1, _S), lambda r: (0, 0)),
            pl.BlockSpec((_HD, 1), lambda r: (0, 0)),
        ],
        out_specs=[
            pl.BlockSpec(memory_space=pl.ANY),
            pl.BlockSpec((1, _HD, _S), lambda r: (0, 0, 0)),
            pl.BlockSpec((1, _HD, _S), lambda r: (0, 0, 0)),
        ],
        out_shape=[
            jax.ShapeDtypeStruct((_B, 1, _S, _S), jnp.float32),
            jax.ShapeDtypeStruct((1, _HD, _S), jnp.float32),
            jax.ShapeDtypeStruct((1, _HD, _S), jnp.float32),
        ],
        scratch_shapes=[
            pltpu.VMEM((2 * _B, _RBLK, _S), jnp.float32),
            pltpu.SemaphoreType.DMA((2 * _B,)),
        ],
    )(attention_mask, position_ids, inv2)


def kernel(input_ids, attention_mask, position_ids, labels, sample_weights, W):
    mask4d, cos_t, sin_t = _mask_rope(attention_mask, position_ids, jnp.asarray(_INV2))
    idx3 = input_ids.reshape(_NW, _NCH, _CH)
    hidden = _make_sc_gather()(W, idx3).reshape(_B, _S, _D)
    cos3 = jnp.transpose(cos_t, (0, 2, 1))
    sin3 = jnp.transpose(sin_t, (0, 2, 1))
    return (hidden, mask4d, cos3, sin3, labels, sample_weights)


# speedup vs baseline: 1.0041x; 1.0041x over previous
import functools

import numpy as np
import jax
import jax.numpy as jnp
from jax import lax
from jax.experimental import pallas as pl
from jax.experimental.pallas import tpu as pltpu
from jax.experimental.pallas import tpu_sc as plsc

_VOCAB = 100000
_D = 1024
_HD = 64
_THETA = 10000.0
_B = 4
_S = 2048
_MIN = float(np.finfo(np.float32).min)

_NC = 2
_NS = 16
_NW = _NC * _NS
_TOK = _B * _S
_RPW = _TOK // _NW
_CH = 32
_NCH = _RPW // _CH


@functools.cache
def _make_sc_gather():
    mesh = plsc.VectorSubcoreMesh(core_axis_name="c", subcore_axis_name="s")

    @functools.partial(
        pl.kernel,
        mesh=mesh,
        out_type=jax.ShapeDtypeStruct((_TOK, _D), jnp.float32),
        scratch_types=[
            pltpu.VMEM((_NCH, _CH), jnp.int32),
            pltpu.VMEM((_CH, _D), jnp.float32),
            pltpu.VMEM((_CH, _D), jnp.float32),
            pltpu.SemaphoreType.DMA,
            pltpu.SemaphoreType.DMA,
            pltpu.SemaphoreType.DMA,
            pltpu.SemaphoreType.DMA,
        ],
    )
    def _sc_gather(table, ids, out, idx_v, buf_a, buf_b, gsem_a, gsem_b, wsem_a, wsem_b):
        wid = lax.axis_index("s") * _NC + lax.axis_index("c")
        base = pl.multiple_of(wid * _RPW, _RPW)
        pltpu.sync_copy(ids.at[wid], idx_v)

        def gather(c, buf, gsem):
            pltpu.async_copy(table.at[idx_v.at[c]], buf, gsem)

        def write(c, buf, wsem):
            pltpu.async_copy(buf, out.at[pl.ds(base + c * _CH, _CH)], wsem)

        def wait_gather(buf, gsem):
            pltpu.make_async_copy(table.at[idx_v.at[0]], buf, gsem).wait()

        def wait_write(buf, wsem):
            pltpu.make_async_copy(buf, out.at[pl.ds(base, _CH)], wsem).wait()

        gather(0, buf_a, gsem_a)
        gather(1, buf_b, gsem_b)
        npair = _NCH // 2

        def body(t, carry):
            c = pl.multiple_of(t * 2, 2)
            wait_gather(buf_a, gsem_a)
            write(c, buf_a, wsem_a)
            wait_gather(buf_b, gsem_b)
            write(c + 1, buf_b, wsem_b)

            @pl.when(t + 1 < npair)
            def _():
                wait_write(buf_a, wsem_a)
                gather(c + 2, buf_a, gsem_a)
                wait_write(buf_b, wsem_b)
                gather(c + 3, buf_b, gsem_b)

            return carry

        lax.fori_loop(0, npair, body, 0)
        wait_write(buf_a, wsem_a)
        wait_write(buf_b, wsem_b)

    return _sc_gather


_RBLK = 256
_NRB = _S // _RBLK

_inv_half = 1.0 / (_THETA ** (np.arange(0, _HD, 2, dtype=np.float32) / np.float32(_HD)))
_INV2 = np.concatenate([_inv_half, _inv_half]).reshape(_HD, 1).astype(np.float32)


def _mask_rope_body(am_ref, pos_ref, inv_ref, mask_hbm, cos_ref, sin_ref, bufs, sems):
    r = pl.program_id(0)
    row = lax.broadcasted_iota(jnp.int32, (_RBLK, _S), 0)
    col = lax.broadcasted_iota(jnp.int32, (_RBLK, _S), 1)
    causal = jnp.where(col - row > r * _RBLK, _MIN, 0.0)
    s = r % 2
    for b in range(_B):
        i = s * _B + b

        @pl.when(r >= 2)
        def _wait(i=i, b=b):
            pltpu.make_async_copy(
                bufs.at[i], mask_hbm.at[b, 0, pl.ds(0, _RBLK)], sems.at[i]
            ).wait()

        pb = jnp.where(am_ref[b, :][None, :] == 0.0, _MIN, 0.0)
        bufs[i] = jnp.minimum(causal, pb)
        pltpu.async_copy(
            bufs.at[i], mask_hbm.at[b, 0, pl.ds(r * _RBLK, _RBLK)], sems.at[i]
        )

    @pl.when(r == _NRB - 1)
    def _drain():
        for i in range(2 * _B):
            pltpu.make_async_copy(
                bufs.at[i], mask_hbm.at[0, 0, pl.ds(0, _RBLK)], sems.at[i]
            ).wait()

    @pl.when(r == 0)
    def _():
        pos_f = pos_ref[...].astype(jnp.float32)
        emb_t = inv_ref[...] * pos_f
        cos_ref[0] = jnp.cos(emb_t)
        sin_ref[0] = jnp.sin(emb_t)


def _mask_rope(attention_mask, position_ids, inv2):
    return pl.pallas_call(
        _mask_rope_body,
        grid=(_NRB,),
        in_specs=[
            pl.BlockSpec((_B, _S), lambda r: (0, 0)),
            pl.BlockSpec((1, _S), lambda r: (0, 0)),
            pl.BlockSpec((_HD, 1), lambda r: (0, 0)),
        ],
        out_specs=[
            pl.BlockSpec(memory_space=pl.ANY),
            pl.BlockSpec((1, _HD, _S), lambda r: (0, 0, 0)),
            pl.BlockSpec((1, _HD, _S), lambda r: (0, 0, 0)),
        ],
        out_shape=[
            jax.ShapeDtypeStruct((_B, 1, _S, _S), jnp.float32),
            jax.ShapeDtypeStruct((1, _HD, _S), jnp.float32),
            jax.ShapeDtypeStruct((1, _HD, _S), jnp.float32),
        ],
        scratch_shapes=[
            pltpu.VMEM((2 * _B, _RBLK, _S), jnp.float32),
            pltpu.SemaphoreType.DMA((2 * _B,)),
        ],
    )(attention_mask, position_ids, inv2)


def kernel(input_ids, attention_mask, position_ids, labels, sample_weights, W):
    mask4d, cos_t, sin_t = _mask_rope(attention_mask, position_ids, jnp.asarray(_INV2))
    idx3 = input_ids.reshape(_NW, _NCH, _CH)
    hidden = _make_sc_gather()(W, idx3).reshape(_B, _S, _D)
    cos3 = jnp.transpose(cos_t, (0, 2, 1))
    sin3 = jnp.transpose(sin_t, (0, 2, 1))
    return (hidden, mask4d, cos3, sin3, labels, sample_weights)


# speedup vs baseline: 1.0214x; 1.0173x over previous
import functools

import numpy as np
import jax
import jax.numpy as jnp
from jax import lax
from jax.experimental import pallas as pl
from jax.experimental.pallas import tpu as pltpu
from jax.experimental.pallas import tpu_sc as plsc

_VOCAB = 100000
_D = 1024
_HD = 64
_THETA = 10000.0
_B = 4
_S = 2048
_MIN = float(np.finfo(np.float32).min)

_NC = 2
_NS = 16
_NW = _NC * _NS
_TOK = _B * _S
_RPW = _TOK // _NW
_CH = 32
_NCH = _RPW // _CH


@functools.cache
def _make_sc_gather():
    mesh = plsc.VectorSubcoreMesh(core_axis_name="c", subcore_axis_name="s")

    @functools.partial(
        pl.kernel,
        mesh=mesh,
        out_type=jax.ShapeDtypeStruct((_TOK, _D), jnp.float32),
        scratch_types=[
            pltpu.VMEM((_NCH, _CH), jnp.int32),
            pltpu.VMEM((_CH, _D), jnp.float32),
            pltpu.VMEM((_CH, _D), jnp.float32),
            pltpu.SemaphoreType.DMA,
            pltpu.SemaphoreType.DMA,
            pltpu.SemaphoreType.DMA,
            pltpu.SemaphoreType.DMA,
        ],
    )
    def _sc_gather(table, ids, out, idx_v, buf_a, buf_b, gsem_a, gsem_b, wsem_a, wsem_b):
        wid = lax.axis_index("s") * _NC + lax.axis_index("c")
        base = pl.multiple_of(wid * _RPW, _RPW)
        pltpu.sync_copy(ids.at[wid], idx_v)

        def gather(c, buf, gsem):
            pltpu.async_copy(table.at[idx_v.at[c]], buf, gsem)

        def write(c, buf, wsem):
            pltpu.async_copy(buf, out.at[pl.ds(base + c * _CH, _CH)], wsem)

        def wait_gather(buf, gsem):
            pltpu.make_async_copy(table.at[idx_v.at[0]], buf, gsem).wait()

        def wait_write(buf, wsem):
            pltpu.make_async_copy(buf, out.at[pl.ds(base, _CH)], wsem).wait()

        gather(0, buf_a, gsem_a)
        gather(1, buf_b, gsem_b)
        npair = _NCH // 2

        def body(t, carry):
            c = pl.multiple_of(t * 2, 2)
            wait_gather(buf_a, gsem_a)
            write(c, buf_a, wsem_a)
            wait_gather(buf_b, gsem_b)
            write(c + 1, buf_b, wsem_b)

            @pl.when(t + 1 < npair)
            def _():
                wait_write(buf_a, wsem_a)
                gather(c + 2, buf_a, gsem_a)
                wait_write(buf_b, wsem_b)
                gather(c + 3, buf_b, gsem_b)

            return carry

        lax.fori_loop(0, npair, body, 0)
        wait_write(buf_a, wsem_a)
        wait_write(buf_b, wsem_b)

    return _sc_gather


_RBLK = 256
_NRB = _S // _RBLK

_inv_half = 1.0 / (_THETA ** (np.arange(0, _HD, 2, dtype=np.float32) / np.float32(_HD)))
_INV2 = np.concatenate([_inv_half, _inv_half]).reshape(_HD, 1).astype(np.float32)


def _mask_rope_body(am_ref, pos_ref, inv_ref, lab_ref, sw_ref,
                    mask_hbm, cos_ref, sin_ref, lab_out, sw_out, bufs, sems):
    r = pl.program_id(0)
    row = lax.broadcasted_iota(jnp.int32, (_RBLK, _S), 0)
    col = lax.broadcasted_iota(jnp.int32, (_RBLK, _S), 1)
    causal = jnp.where(col - row > r * _RBLK, _MIN, 0.0)
    s = r % 2
    for b in range(_B):
        i = s * _B + b

        @pl.when(r >= 2)
        def _wait(i=i, b=b):
            pltpu.make_async_copy(
                bufs.at[i], mask_hbm.at[b, 0, pl.ds(0, _RBLK)], sems.at[i]
            ).wait()

        pb = jnp.where(am_ref[b, :][None, :] == 0.0, _MIN, 0.0)
        bufs[i] = jnp.minimum(causal, pb)
        pltpu.async_copy(
            bufs.at[i], mask_hbm.at[b, 0, pl.ds(r * _RBLK, _RBLK)], sems.at[i]
        )

    @pl.when(r == _NRB - 1)
    def _drain():
        for i in range(2 * _B):
            pltpu.make_async_copy(
                bufs.at[i], mask_hbm.at[0, 0, pl.ds(0, _RBLK)], sems.at[i]
            ).wait()

    @pl.when(r == 0)
    def _():
        pos_f = pos_ref[...].astype(jnp.float32)
        emb_t = inv_ref[...] * pos_f
        cos_ref[0] = jnp.cos(emb_t)
        sin_ref[0] = jnp.sin(emb_t)
        lab_out[...] = lab_ref[...]
        sw_out[...] = sw_ref[...]


def _mask_rope(attention_mask, position_ids, inv2, labels, sample_weights):
    return pl.pallas_call(
        _mask_rope_body,
        grid=(_NRB,),
        in_specs=[
            pl.BlockSpec((_B, _S), lambda r: (0, 0)),
            pl.BlockSpec((1, _S), lambda r: (0, 0)),
            pl.BlockSpec((_HD, 1), lambda r: (0, 0)),
            pl.BlockSpec((_B, _S), lambda r: (0, 0)),
            pl.BlockSpec((1, _B), lambda r: (0, 0)),
        ],
        out_specs=[
            pl.BlockSpec(memory_space=pl.ANY),
            pl.BlockSpec((1, _HD, _S), lambda r: (0, 0, 0)),
            pl.BlockSpec((1, _HD, _S), lambda r: (0, 0, 0)),
            pl.BlockSpec((_B, _S), lambda r: (0, 0)),
            pl.BlockSpec((1, _B), lambda r: (0, 0)),
        ],
        out_shape=[
            jax.ShapeDtypeStruct((_B, 1, _S, _S), jnp.float32),
            jax.ShapeDtypeStruct((1, _HD, _S), jnp.float32),
            jax.ShapeDtypeStruct((1, _HD, _S), jnp.float32),
            jax.ShapeDtypeStruct((_B, _S), jnp.int32),
            jax.ShapeDtypeStruct((1, _B), jnp.float32),
        ],
        scratch_shapes=[
            pltpu.VMEM((2 * _B, _RBLK, _S), jnp.float32),
            pltpu.SemaphoreType.DMA((2 * _B,)),
        ],
    )(attention_mask, position_ids, inv2, labels, sample_weights)


def kernel(input_ids, attention_mask, position_ids, labels, sample_weights, W):
    mask4d, cos_t, sin_t, lab_out, sw_out = _mask_rope(
        attention_mask, position_ids, jnp.asarray(_INV2),
        labels, sample_weights.reshape(1, _B))
    idx3 = input_ids.reshape(_NW, _NCH, _CH)
    hidden = _make_sc_gather()(W, idx3).reshape(_B, _S, _D)
    cos3 = jnp.transpose(cos_t, (0, 2, 1))
    sin3 = jnp.transpose(sin_t, (0, 2, 1))
    return (hidden, mask4d, cos3, sin3, lab_out, sw_out.reshape(_B))
